# baseline (speedup 1.0000x reference)
import functools

import jax
import jax.numpy as jnp
from jax import lax
from jax.experimental import pallas as pl
from jax.experimental.pallas import tpu as pltpu
from jax.experimental.pallas import tpu_sc as plsc

N = 16384
C = 5000
NUM_CORES = 2
SUBCORES = 16
NUM_WORKERS = NUM_CORES * SUBCORES
PER_WORKER = N // NUM_WORKERS
WIN = 128
NUM_WIN = PER_WORKER // WIN
LANES = 16


def _loss_kernel(probT_hbm, tgt_hbm, rew_hbm, part_hbm,
                 tgt_v, rew_v, dst0_v, dst1_v, dst2_v, dst3_v,
                 acc_v, sem0, sem1, sem2, sem3, semt, semr):
    cid = lax.axis_index("c")
    sid = lax.axis_index("s")
    wid = cid * SUBCORES + sid
    base = wid * PER_WORKER

    cpts = [
        pltpu.async_copy(
            tgt_hbm.at[pl.ds(base + s * WIN, WIN)],
            tgt_v.at[pl.ds(s * WIN, WIN)], (sem0, sem1, sem2, sem3)[s])
        for s in range(NUM_WIN)
    ]
    del semt
    cpr = pltpu.async_copy(rew_hbm.at[pl.ds(base, PER_WORKER)], rew_v, semr)

    dsts = (dst0_v, dst1_v, dst2_v, dst3_v)
    sems = (sem0, sem1, sem2, sem3)

    copies = []
    for s in range(NUM_WIN):
        cpts[s].wait()
        copies.append(pltpu.async_copy(
            probT_hbm.at[tgt_v.at[pl.ds(s * WIN, WIN)],
                         pl.ds(base + s * WIN, WIN)],
            dsts[s], sems[s]))
    cpr.wait()

    lane = lax.iota(jnp.int32, LANES)
    acc = jnp.zeros((LANES,), jnp.float32)
    for s in range(NUM_WIN):
        copies[s].wait()
        for g in range(WIN // LANES):
            kk = g * LANES + lane
            picked = plsc.load_gather(dsts[s], [kk, kk])
            acc = acc + picked * rew_v[pl.ds(s * WIN + g * LANES, LANES)]
    acc_v[...] = -acc

    pltpu.sync_copy(acc_v, part_hbm.at[wid])


@jax.jit
def _loss(probT, target, reward):
    mesh = plsc.VectorSubcoreMesh(core_axis_name="c", subcore_axis_name="s",
                                  num_cores=NUM_CORES)
    k = functools.partial(
        pl.kernel,
        mesh=mesh,
        out_type=jax.ShapeDtypeStruct((NUM_WORKERS, LANES), jnp.float32),
        scratch_types=[
            pltpu.VMEM((PER_WORKER,), jnp.int32),
            pltpu.VMEM((PER_WORKER,), jnp.float32),
            pltpu.VMEM((WIN, WIN), jnp.float32),
            pltpu.VMEM((WIN, WIN), jnp.float32),
            pltpu.VMEM((WIN, WIN), jnp.float32),
            pltpu.VMEM((WIN, WIN), jnp.float32),
            pltpu.VMEM((LANES,), jnp.float32),
            pltpu.SemaphoreType.DMA,
            pltpu.SemaphoreType.DMA,
            pltpu.SemaphoreType.DMA,
            pltpu.SemaphoreType.DMA,
            pltpu.SemaphoreType.DMA,
            pltpu.SemaphoreType.DMA,
        ],
        compiler_params=pltpu.CompilerParams(
            needs_layout_passes=False,
            skip_device_barrier=True,
        ),
    )(_loss_kernel)
    return k(probT, target, reward)


def kernel(prob, target, reward):
    part = _loss(prob.T, target.astype(jnp.int32), reward)
    return jnp.sum(part)
